# baseline (speedup 1.0000x reference)
import functools

import jax
import jax.numpy as jnp
from jax import lax
from jax.experimental import pallas as pl
from jax.experimental.pallas import tpu as pltpu
from jax.experimental.pallas import tpu_sc as plsc

L = 16
NC = 2
NS = 16
NW = NC * NS
TRASH = 262144
K_ROUNDS = 6
CAP = 1024

_SC_PARAMS = pltpu.CompilerParams(use_tc_tiling_on_sc=False)


def _fill_positions(posv_v, base, n):
    @plsc.parallel_loop(0, n // L, unroll=8)
    def _(j):
        off = pl.multiple_of(j * L, L)
        posv_v[pl.ds(off, L)] = base + j * L + lax.iota(jnp.int32, 16)


def _make_gather_round0(M, B, D):
    ch = B // NW
    mesh = plsc.VectorSubcoreMesh(core_axis_name="c", subcore_axis_name="s")

    @functools.partial(
        pl.kernel,
        out_type=jax.ShapeDtypeStruct((B, D), jnp.float32),
        mesh=mesh,
        compiler_params=_SC_PARAMS,
        scratch_types=[
            pltpu.VMEM((ch,), jnp.int32),
            pltpu.VMEM((ch, D), jnp.float32),
            pltpu.VMEM((ch,), jnp.int32),
            pltpu.SemaphoreType.DMA,
            pltpu.SemaphoreType.DMA,
        ],
    )
    def gather_k(mem_hbm, idx_hbm, pos_hbm, ext_hbm, idx_v, rows_v,
                 posv_v, sem_g, sem_s):
        wid = lax.axis_index("s") * NC + lax.axis_index("c")
        base = wid * ch
        pltpu.sync_copy(idx_hbm.at[pl.ds(base, ch)], idx_v)
        _fill_positions(posv_v, base, ch)
        g = pltpu.async_copy(mem_hbm.at[idx_v], rows_v, sem_g)
        s = pltpu.async_copy(posv_v, pos_hbm.at[idx_v], sem_s)
        g.wait()
        pltpu.sync_copy(rows_v, ext_hbm.at[pl.ds(base, ch)])
        s.wait()

    return gather_k


def _make_round(M, B):
    ch = B // NW
    mesh = plsc.VectorSubcoreMesh(core_axis_name="c", subcore_axis_name="s")

    nchunk = 4
    h = ch // nchunk

    @functools.partial(
        pl.kernel,
        mesh=mesh,
        compiler_params=_SC_PARAMS,
        scratch_types=[
            pltpu.VMEM((ch,), jnp.int32),
            pltpu.VMEM((ch,), jnp.int32),
            pltpu.VMEM((ch,), jnp.int32),
            pltpu.VMEM((ch,), jnp.int32),
        ] + [pltpu.SemaphoreType.DMA] * (2 * nchunk),
    )
    def round_k(idx_hbm, pos_hbm, idx_v, w_v, posv_v, idx2_v, *sems):
        wid = lax.axis_index("s") * NC + lax.axis_index("c")
        base = wid * ch
        pltpu.sync_copy(idx_hbm.at[pl.ds(base, ch)], idx_v)
        _fill_positions(posv_v, base, ch)
        gs = [
            pltpu.async_copy(pos_hbm.at[idx_v.at[pl.ds(k * h, h)]],
                             w_v.at[pl.ds(k * h, h)], sems[k])
            for k in range(nchunk)
        ]
        ss = []
        for k in range(nchunk):
            gs[k].wait()

            @plsc.parallel_loop(k * (h // L), (k + 1) * (h // L), unroll=8)
            def _(j):
                off = pl.multiple_of(j * L, L)
                w16 = w_v[pl.ds(off, L)]
                p16 = posv_v[pl.ds(off, L)]
                i16 = idx_v[pl.ds(off, L)]
                pend = w16 < p16
                t16 = M + p16
                idx2_v[pl.ds(off, L)] = jnp.where(pend, i16, t16)

            ss.append(
                pltpu.async_copy(posv_v.at[pl.ds(k * h, h)],
                                 pos_hbm.at[idx2_v.at[pl.ds(k * h, h)]],
                                 sems[nchunk + k]))
        for s in ss:
            s.wait()

    return round_k


def _make_apply(M, B, D):
    ch = B // NW
    mesh = plsc.VectorSubcoreMesh(core_axis_name="c", subcore_axis_name="s")

    @functools.partial(
        pl.kernel,
        mesh=mesh,
        compiler_params=_SC_PARAMS,
        scratch_types=[
            pltpu.VMEM((ch,), jnp.int32),
            pltpu.VMEM((ch,), jnp.int32),
            pltpu.VMEM((ch, D), jnp.float32),
            pltpu.SemaphoreType.DMA,
            pltpu.SemaphoreType.DMA,
        ],
    )
    def apply_k(idx_hbm, pos_hbm, fused_hbm, out_hbm, idx_v, w_v, rows_v,
                sem_g, sem_s):
        wid = lax.axis_index("s") * NC + lax.axis_index("c")
        base = wid * ch
        pltpu.sync_copy(idx_hbm.at[pl.ds(base, ch)], idx_v)
        pltpu.async_copy(pos_hbm.at[idx_v], w_v, sem_g).wait()
        pltpu.async_copy(fused_hbm.at[w_v], rows_v, sem_g).wait()
        pltpu.async_copy(rows_v, out_hbm.at[idx_v], sem_s).wait()

    return apply_k


def _copy_body(m_ref, o_ref):
    o_ref[...] = m_ref[...]


def _copy_mem(mem):
    M, D = mem.shape
    n = 512
    return pl.pallas_call(
        _copy_body,
        grid=(n,),
        in_specs=[pl.BlockSpec((M // n, D), lambda i: (i, 0))],
        out_specs=pl.BlockSpec((M // n, D), lambda i: (i, 0)),
        out_shape=jax.ShapeDtypeStruct((M, D), jnp.float32),
    )(mem)


def _fusion_body(cv_ref, ext_ref, wc_ref, we_ref, b_ref, o_ref):
    acc = jnp.dot(cv_ref[...], wc_ref[...], preferred_element_type=jnp.float32)
    acc = acc + jnp.dot(ext_ref[...], we_ref[...],
                        preferred_element_type=jnp.float32)
    o_ref[...] = jnp.tanh(acc + b_ref[...])


def _fusion(cost_volume, ext, W, b, B, C, D):
    gpr = 16
    rows = B // gpr
    blk = 2048
    cv2 = cost_volume.reshape(rows, gpr * C)
    ext2 = ext.reshape(rows, gpr * D)
    eye = jnp.eye(gpr, dtype=W.dtype)
    wc_big = jnp.kron(eye, W[:C])
    we_big = jnp.kron(eye, W[C:])
    bt = jnp.tile(b, gpr).reshape(1, gpr * D)
    out2 = pl.pallas_call(
        _fusion_body,
        grid=(rows // blk,),
        in_specs=[
            pl.BlockSpec((blk, gpr * C), lambda i: (i, 0)),
            pl.BlockSpec((blk, gpr * D), lambda i: (i, 0)),
            pl.BlockSpec((gpr * C, gpr * D), lambda i: (0, 0)),
            pl.BlockSpec((gpr * D, gpr * D), lambda i: (0, 0)),
            pl.BlockSpec((1, gpr * D), lambda i: (0, 0)),
        ],
        out_specs=pl.BlockSpec((blk, gpr * D), lambda i: (i, 0)),
        out_shape=jax.ShapeDtypeStruct((rows, gpr * D), jnp.float32),
    )(cv2, ext2, wc_big, we_big, bt)
    return out2.reshape(B, D)


def kernel(mem, cost_volume, W, b, indices):
    M, D = mem.shape
    B, C = cost_volume.shape
    idx = indices.astype(jnp.int32)

    pos_ref = jax.new_ref(jnp.zeros((M + TRASH,), jnp.int32))
    ext = _make_gather_round0(M, B, D)(mem, idx, pos_ref)
    fused = _fusion(cost_volume, ext, W, b, B, C, D)
    round_k = _make_round(M, B)
    for _ in range(K_ROUNDS):
        round_k(idx, pos_ref)
    out_ref = jax.new_ref(_copy_mem(mem))
    _make_apply(M, B, D)(idx, pos_ref, fused, out_ref)
    return out_ref[...]


# speedup vs baseline: 1.0842x; 1.0842x over previous
import functools

import jax
import jax.numpy as jnp
from jax import lax
from jax.experimental import pallas as pl
from jax.experimental.pallas import tpu as pltpu
from jax.experimental.pallas import tpu_sc as plsc

L = 16
NC = 2
NS = 16
NW = NC * NS
TRASH = 262144
K_ROUNDS = 6
CAP = 1024

_SC_PARAMS = pltpu.CompilerParams(use_tc_tiling_on_sc=False)


def _fill_positions(posv_v, base, n):
    @plsc.parallel_loop(0, n // L, unroll=8)
    def _(j):
        off = pl.multiple_of(j * L, L)
        posv_v[pl.ds(off, L)] = base + j * L + lax.iota(jnp.int32, 16)


def _make_gather_round0(M, B, D):
    ch = B // NW
    mesh = plsc.VectorSubcoreMesh(core_axis_name="c", subcore_axis_name="s")

    @functools.partial(
        pl.kernel,
        out_type=jax.ShapeDtypeStruct((B, D), jnp.float32),
        mesh=mesh,
        compiler_params=_SC_PARAMS,
        scratch_types=[
            pltpu.VMEM((ch,), jnp.int32),
            pltpu.VMEM((ch, D), jnp.float32),
            pltpu.VMEM((ch,), jnp.int32),
            pltpu.SemaphoreType.DMA,
            pltpu.SemaphoreType.DMA,
        ],
    )
    def gather_k(mem_hbm, idx_hbm, pos_hbm, ext_hbm, idx_v, rows_v,
                 posv_v, sem_g, sem_s):
        wid = lax.axis_index("s") * NC + lax.axis_index("c")
        base = wid * ch
        pltpu.sync_copy(idx_hbm.at[pl.ds(base, ch)], idx_v)
        _fill_positions(posv_v, base, ch)
        g = pltpu.async_copy(mem_hbm.at[idx_v], rows_v, sem_g)
        s = pltpu.async_copy(posv_v, pos_hbm.at[idx_v], sem_s)
        g.wait()
        pltpu.sync_copy(rows_v, ext_hbm.at[pl.ds(base, ch)])
        s.wait()

    return gather_k


def _make_round(M, B):
    ch = B // NW
    mesh = plsc.VectorSubcoreMesh(core_axis_name="c", subcore_axis_name="s")

    nchunk = 4
    h = ch // nchunk

    @functools.partial(
        pl.kernel,
        mesh=mesh,
        compiler_params=_SC_PARAMS,
        scratch_types=[
            pltpu.VMEM((ch,), jnp.int32),
            pltpu.VMEM((ch,), jnp.int32),
            pltpu.VMEM((ch,), jnp.int32),
            pltpu.VMEM((ch,), jnp.int32),
        ] + [pltpu.SemaphoreType.DMA] * (2 * nchunk),
    )
    def round_k(idx_hbm, pos_hbm, idx_v, w_v, posv_v, idx2_v, *sems):
        wid = lax.axis_index("s") * NC + lax.axis_index("c")
        base = wid * ch
        pltpu.sync_copy(idx_hbm.at[pl.ds(base, ch)], idx_v)
        _fill_positions(posv_v, base, ch)
        gs = [
            pltpu.async_copy(pos_hbm.at[idx_v.at[pl.ds(k * h, h)]],
                             w_v.at[pl.ds(k * h, h)], sems[k])
            for k in range(nchunk)
        ]
        ss = []
        for k in range(nchunk):
            gs[k].wait()

            @plsc.parallel_loop(k * (h // L), (k + 1) * (h // L), unroll=8)
            def _(j):
                off = pl.multiple_of(j * L, L)
                w16 = w_v[pl.ds(off, L)]
                p16 = posv_v[pl.ds(off, L)]
                i16 = idx_v[pl.ds(off, L)]
                pend = w16 < p16
                t16 = M + p16
                idx2_v[pl.ds(off, L)] = jnp.where(pend, i16, t16)

            ss.append(
                pltpu.async_copy(posv_v.at[pl.ds(k * h, h)],
                                 pos_hbm.at[idx2_v.at[pl.ds(k * h, h)]],
                                 sems[nchunk + k]))
        for s in ss:
            s.wait()

    return round_k


def _make_apply(M, B, D):
    ch = B // NW
    mesh = plsc.VectorSubcoreMesh(core_axis_name="c", subcore_axis_name="s")

    @functools.partial(
        pl.kernel,
        mesh=mesh,
        compiler_params=_SC_PARAMS,
        scratch_types=[
            pltpu.VMEM((ch,), jnp.int32),
            pltpu.VMEM((ch,), jnp.int32),
            pltpu.VMEM((ch, D), jnp.float32),
            pltpu.SemaphoreType.DMA,
            pltpu.SemaphoreType.DMA,
        ],
    )
    def apply_k(idx_hbm, pos_hbm, fused_hbm, out_hbm, idx_v, w_v, rows_v,
                sem_g, sem_s):
        wid = lax.axis_index("s") * NC + lax.axis_index("c")
        base = wid * ch
        pltpu.sync_copy(idx_hbm.at[pl.ds(base, ch)], idx_v)
        pltpu.async_copy(pos_hbm.at[idx_v], w_v, sem_g).wait()
        pltpu.async_copy(fused_hbm.at[w_v], rows_v, sem_g).wait()
        pltpu.async_copy(rows_v, out_hbm.at[idx_v], sem_s).wait()

    return apply_k


def _copy_body(m_ref, o_ref):
    o_ref[...] = m_ref[...]


def _copy_mem(mem):
    M, D = mem.shape
    n = 512
    return pl.pallas_call(
        _copy_body,
        grid=(n,),
        in_specs=[pl.BlockSpec((M // n, D), lambda i: (i, 0))],
        out_specs=pl.BlockSpec((M // n, D), lambda i: (i, 0)),
        out_shape=jax.ShapeDtypeStruct((M, D), jnp.float32),
    )(mem)


def _fusion_body(cv_ref, ext_ref, wc_ref, we_ref, b_ref, o_ref):
    acc = jnp.dot(cv_ref[...], wc_ref[...], preferred_element_type=jnp.float32)
    acc = acc + jnp.dot(ext_ref[...], we_ref[...],
                        preferred_element_type=jnp.float32)
    o_ref[...] = jnp.tanh(acc + b_ref[...])


def _fusion(cost_volume, ext, W, b, B, C, D):
    gpr = 16
    rows = B // gpr
    blk = 2048
    cv2 = cost_volume.reshape(rows, gpr * C)
    ext2 = ext.reshape(rows, gpr * D)
    eye = jnp.eye(gpr, dtype=W.dtype)
    wc_big = jnp.kron(eye, W[:C])
    we_big = jnp.kron(eye, W[C:])
    bt = jnp.tile(b, gpr).reshape(1, gpr * D)
    out2 = pl.pallas_call(
        _fusion_body,
        grid=(rows // blk,),
        in_specs=[
            pl.BlockSpec((blk, gpr * C), lambda i: (i, 0)),
            pl.BlockSpec((blk, gpr * D), lambda i: (i, 0)),
            pl.BlockSpec((gpr * C, gpr * D), lambda i: (0, 0)),
            pl.BlockSpec((gpr * D, gpr * D), lambda i: (0, 0)),
            pl.BlockSpec((1, gpr * D), lambda i: (0, 0)),
        ],
        out_specs=pl.BlockSpec((blk, gpr * D), lambda i: (i, 0)),
        out_shape=jax.ShapeDtypeStruct((rows, gpr * D), jnp.float32),
    )(cv2, ext2, wc_big, we_big, bt)
    return out2.reshape(B, D)


def kernel(mem, cost_volume, W, b, indices):
    M, D = mem.shape
    B, C = cost_volume.shape
    idx = indices.astype(jnp.int32)

    pos_ref = jax.new_ref(jnp.zeros((M + TRASH,), jnp.int32))
    ext = _make_gather_round0(M, B, D)(mem, idx, pos_ref)
    fused = _fusion(cost_volume, ext, W, b, B, C, D)
    round_k = _make_round(M, B)
    for _ in range(K_ROUNDS):
        round_k(idx, pos_ref)
    mem_d, fused = lax.optimization_barrier((mem, fused))
    out_ref = jax.new_ref(jnp.copy(mem_d))
    _make_apply(M, B, D)(idx, pos_ref, fused, out_ref)
    return out_ref[...]


# speedup vs baseline: 1.4183x; 1.3082x over previous
import functools

import jax
import jax.numpy as jnp
from jax import lax
from jax.experimental import pallas as pl
from jax.experimental.pallas import tpu as pltpu
from jax.experimental.pallas import tpu_sc as plsc

L = 16
NC = 2
NS = 16
NW = NC * NS
TRASH = 262144
K_ROUNDS = 6

_SC_PARAMS = pltpu.CompilerParams(use_tc_tiling_on_sc=False)


def _fill_positions(posv_v, base, n):
    @plsc.parallel_loop(0, n // L, unroll=8)
    def _(j):
        off = pl.multiple_of(j * L, L)
        posv_v[pl.ds(off, L)] = base + j * L + lax.iota(jnp.int32, 16)


def _make_gather_round0(M, B, D):
    ch = B // NW
    mesh = plsc.VectorSubcoreMesh(core_axis_name="c", subcore_axis_name="s")

    @functools.partial(
        pl.kernel,
        out_type=jax.ShapeDtypeStruct((B, D), jnp.float32),
        mesh=mesh,
        compiler_params=_SC_PARAMS,
        scratch_types=[
            pltpu.VMEM((ch,), jnp.int32),
            pltpu.VMEM((ch, D), jnp.float32),
            pltpu.VMEM((ch,), jnp.int32),
            pltpu.SemaphoreType.DMA,
            pltpu.SemaphoreType.DMA,
        ],
    )
    def gather_k(mem_hbm, idx_hbm, pos_hbm, ext_hbm, idx_v, rows_v,
                 posv_v, sem_g, sem_s):
        wid = lax.axis_index("s") * NC + lax.axis_index("c")
        base = wid * ch
        pltpu.sync_copy(idx_hbm.at[pl.ds(base, ch)], idx_v)
        _fill_positions(posv_v, base, ch)
        g = pltpu.async_copy(mem_hbm.at[idx_v], rows_v, sem_g)
        s = pltpu.async_copy(posv_v, pos_hbm.at[idx_v], sem_s)
        g.wait()
        pltpu.sync_copy(rows_v, ext_hbm.at[pl.ds(base, ch)])
        s.wait()

    return gather_k


def _make_round(M, B):
    ch = B // NW
    mesh = plsc.VectorSubcoreMesh(core_axis_name="c", subcore_axis_name="s")

    nchunk = 4
    h = ch // nchunk

    @functools.partial(
        pl.kernel,
        mesh=mesh,
        compiler_params=_SC_PARAMS,
        scratch_types=[
            pltpu.VMEM((ch,), jnp.int32),
            pltpu.VMEM((ch,), jnp.int32),
            pltpu.VMEM((ch,), jnp.int32),
            pltpu.VMEM((ch,), jnp.int32),
        ] + [pltpu.SemaphoreType.DMA] * (2 * nchunk),
    )
    def round_k(idx_hbm, pos_hbm, idx_v, w_v, posv_v, idx2_v, *sems):
        wid = lax.axis_index("s") * NC + lax.axis_index("c")
        base = wid * ch
        pltpu.sync_copy(idx_hbm.at[pl.ds(base, ch)], idx_v)
        _fill_positions(posv_v, base, ch)
        gs = [
            pltpu.async_copy(pos_hbm.at[idx_v.at[pl.ds(k * h, h)]],
                             w_v.at[pl.ds(k * h, h)], sems[k])
            for k in range(nchunk)
        ]
        ss = []
        for k in range(nchunk):
            gs[k].wait()

            @plsc.parallel_loop(k * (h // L), (k + 1) * (h // L), unroll=8)
            def _(j):
                off = pl.multiple_of(j * L, L)
                w16 = w_v[pl.ds(off, L)]
                p16 = posv_v[pl.ds(off, L)]
                i16 = idx_v[pl.ds(off, L)]
                pend = w16 < p16
                t16 = M + p16
                idx2_v[pl.ds(off, L)] = jnp.where(pend, i16, t16)

            ss.append(
                pltpu.async_copy(posv_v.at[pl.ds(k * h, h)],
                                 pos_hbm.at[idx2_v.at[pl.ds(k * h, h)]],
                                 sems[nchunk + k]))
        for s in ss:
            s.wait()

    return round_k


def _make_apply(M, B, D):
    ch = B // NW
    mesh = plsc.VectorSubcoreMesh(core_axis_name="c", subcore_axis_name="s")

    @functools.partial(
        pl.kernel,
        mesh=mesh,
        compiler_params=_SC_PARAMS,
        scratch_types=[
            pltpu.VMEM((ch,), jnp.int32),
            pltpu.VMEM((ch,), jnp.int32),
            pltpu.VMEM((ch, D), jnp.float32),
            pltpu.SemaphoreType.DMA,
            pltpu.SemaphoreType.DMA,
        ],
    )
    def apply_k(idx_hbm, pos_hbm, fused_hbm, out_hbm, idx_v, w_v, rows_v,
                sem_g, sem_s):
        wid = lax.axis_index("s") * NC + lax.axis_index("c")
        base = wid * ch
        pltpu.sync_copy(idx_hbm.at[pl.ds(base, ch)], idx_v)
        pltpu.async_copy(pos_hbm.at[idx_v], w_v, sem_g).wait()
        pltpu.async_copy(fused_hbm.at[w_v], rows_v, sem_g).wait()
        pltpu.async_copy(rows_v, out_hbm.at[idx_v], sem_s).wait()

    return apply_k


def _fusion_body(cv_ref, ext_ref, wc_ref, we_ref, b_ref, o_ref):
    acc = jnp.dot(cv_ref[...], wc_ref[...], preferred_element_type=jnp.float32)
    acc = acc + jnp.dot(ext_ref[...], we_ref[...],
                        preferred_element_type=jnp.float32)
    o_ref[...] = jnp.tanh(acc + b_ref[...])


def _fusion(cost_volume, ext, W, b, B, C, D):
    gpr = 16
    rows = B // gpr
    blk = 2048
    cv2 = cost_volume.reshape(rows, gpr * C)
    ext2 = ext.reshape(rows, gpr * D)
    eye = jnp.eye(gpr, dtype=W.dtype)
    wc_big = jnp.kron(eye, W[:C])
    we_big = jnp.kron(eye, W[C:])
    bt = jnp.tile(b, gpr).reshape(1, gpr * D)
    out2 = pl.pallas_call(
        _fusion_body,
        grid=(rows // blk,),
        in_specs=[
            pl.BlockSpec((blk, gpr * C), lambda i: (i, 0)),
            pl.BlockSpec((blk, gpr * D), lambda i: (i, 0)),
            pl.BlockSpec((gpr * C, gpr * D), lambda i: (0, 0)),
            pl.BlockSpec((gpr * D, gpr * D), lambda i: (0, 0)),
            pl.BlockSpec((1, gpr * D), lambda i: (0, 0)),
        ],
        out_specs=pl.BlockSpec((blk, gpr * D), lambda i: (i, 0)),
        out_shape=jax.ShapeDtypeStruct((rows, gpr * D), jnp.float32),
    )(cv2, ext2, wc_big, we_big, bt)
    return out2.reshape(B, D)


def kernel(mem, cost_volume, W, b, indices):
    M, D = mem.shape
    B, C = cost_volume.shape
    idx = indices.astype(jnp.int32)

    pos_ref = jax.new_ref(jnp.zeros((M + TRASH,), jnp.int32))
    ext = _make_gather_round0(M, B, D)(mem, idx, pos_ref)
    fused = _fusion(cost_volume, ext, W, b, B, C, D)
    round_k = _make_round(M, B)
    for _ in range(K_ROUNDS):
        round_k(idx, pos_ref)
    out_ref = jax.new_ref(jnp.copy(mem))
    _make_apply(M, B, D)(idx, pos_ref, fused, out_ref)
    return out_ref[...]
